# baseline (speedup 1.0000x reference)
import jax
import jax.numpy as jnp
from jax.experimental import pallas as pl

B, L, D = 4096, 200, 64
N = B * L
N2 = N // 2

ROWS_PER_BLOCK = 4096
GRID = N2 // ROWS_PER_BLOCK


def _body(mask_ref, w_ref, out_ref):
    m = mask_ref[...]
    me = m[:, 0:1]
    mo = m[:, 1:2]
    i4 = jax.lax.broadcasted_iota(jnp.int32, (1, 4), 1)
    oh = jnp.concatenate(
        [(me == i4).astype(jnp.float32),
         (mo == i4).astype(jnp.float32)], axis=1)
    out_ref[...] = jnp.dot(oh, w_ref[...],
                           preferred_element_type=jnp.float32)


def kernel(mask, emb_weight):
    shape = mask.shape
    flat = mask.reshape(N2, 2).astype(jnp.int32)
    wcat = jnp.zeros((8, 2 * D), jnp.float32)
    wcat = wcat.at[0:4, 0:D].set(emb_weight)
    wcat = wcat.at[4:8, D:2 * D].set(emb_weight)
    out = pl.pallas_call(
        _body,
        grid=(GRID,),
        in_specs=[
            pl.BlockSpec((ROWS_PER_BLOCK, 2), lambda g: (g, 0)),
            pl.BlockSpec((8, 2 * D), lambda g: (0, 0)),
        ],
        out_specs=pl.BlockSpec((ROWS_PER_BLOCK, 2 * D), lambda g: (g, 0)),
        out_shape=jax.ShapeDtypeStruct((N2, 2 * D), jnp.float32),
    )(flat, wcat)
    return out.reshape(shape[0], shape[1], D)


# speedup vs baseline: 1.8491x; 1.8491x over previous
import jax
import jax.numpy as jnp
from jax.experimental import pallas as pl

B, L, D = 4096, 200, 64

BB = 64
GRID = B // BB


def _body(mask_ref, w_ref, out_ref):
    m = mask_ref[...][:, :, None]
    w = w_ref[...]
    w0 = w[0:1, :].reshape(1, 1, D)
    w1 = w[1:2, :].reshape(1, 1, D)
    w2 = w[2:3, :].reshape(1, 1, D)
    w3 = w[3:4, :].reshape(1, 1, D)
    out_ref[...] = jnp.where(m == 0, w0,
                   jnp.where(m == 1, w1,
                   jnp.where(m == 2, w2, w3)))


def kernel(mask, emb_weight):
    return pl.pallas_call(
        _body,
        grid=(GRID,),
        in_specs=[
            pl.BlockSpec((BB, L), lambda g: (g, 0)),
            pl.BlockSpec((4, D), lambda g: (0, 0)),
        ],
        out_specs=pl.BlockSpec((BB, L, D), lambda g: (g, 0, 0)),
        out_shape=jax.ShapeDtypeStruct((B, L, D), jnp.float32),
    )(mask.astype(jnp.int32), emb_weight)


# speedup vs baseline: 1.8852x; 1.0195x over previous
import jax
import jax.numpy as jnp
from jax.experimental import pallas as pl

B, L, D = 4096, 200, 64

BB = 128
GRID = B // BB


def _body(mask_ref, w_ref, out_ref):
    m = mask_ref[...][:, :, None]
    w = w_ref[...]
    w0 = w[0:1, :].reshape(1, 1, D)
    w1 = w[1:2, :].reshape(1, 1, D)
    w2 = w[2:3, :].reshape(1, 1, D)
    w3 = w[3:4, :].reshape(1, 1, D)
    out_ref[...] = jnp.where(m == 0, w0,
                   jnp.where(m == 1, w1,
                   jnp.where(m == 2, w2, w3)))


def kernel(mask, emb_weight):
    return pl.pallas_call(
        _body,
        grid=(GRID,),
        in_specs=[
            pl.BlockSpec((BB, L), lambda g: (g, 0)),
            pl.BlockSpec((4, D), lambda g: (0, 0)),
        ],
        out_specs=pl.BlockSpec((BB, L, D), lambda g: (g, 0, 0)),
        out_shape=jax.ShapeDtypeStruct((B, L, D), jnp.float32),
    )(mask.astype(jnp.int32), emb_weight)
